# baseline (speedup 1.0000x reference)
import jax
import jax.numpy as jnp
from jax.experimental import pallas as pl
from jax.experimental.pallas import tpu as pltpu
from jax.experimental.pallas import tpu_sc as plsc

EMB = 32
MLP_EMB = 128
PACK = MLP_EMB // EMB
GATHER_WINDOW = 128
B_BLK = 2048


def _sc_gather(uq2, iq2, u2, i2, gmf_user_p, gmf_item_p, mlp_user_w, mlp_item_w):
    n = u2.shape[1]
    mesh = plsc.VectorSubcoreMesh(core_axis_name="core", subcore_axis_name="subcore")
    out_types = (
        jax.ShapeDtypeStruct((n, MLP_EMB), jnp.float32),
        jax.ShapeDtypeStruct((n, MLP_EMB), jnp.float32),
        jax.ShapeDtypeStruct((n, MLP_EMB), jnp.float32),
        jax.ShapeDtypeStruct((n, MLP_EMB), jnp.float32),
    )

    @pl.kernel(out_type=out_types, mesh=mesh)
    def gather_kernel(uq_hbm, iq_hbm, u_hbm, i_hbm,
                      gu_w_hbm, gi_w_hbm, mu_w_hbm, mi_w_hbm,
                      gu_hbm, gi_hbm, mu_hbm, mi_hbm):
        idx_spec = pl.BlockSpec((1, GATHER_WINDOW), lambda i: (0, i))
        row_spec = pl.BlockSpec((GATHER_WINDOW, MLP_EMB), lambda i: (i, 0))

        def mlp_gather_body(u_vmem, i_vmem, mu_vmem, mi_vmem):
            pltpu.sync_copy(mu_w_hbm.at[u_vmem.at[0]], mu_vmem)
            pltpu.sync_copy(mi_w_hbm.at[i_vmem.at[0]], mi_vmem)

        pltpu.emit_pipeline(
            mlp_gather_body,
            grid=(n // GATHER_WINDOW,),
            in_specs=[idx_spec, idx_spec],
            out_specs=[row_spec, row_spec],
            core_axis_name=("core", "subcore"),
            dimension_semantics=(pltpu.PARALLEL,),
        )(u_hbm, i_hbm, mu_hbm, mi_hbm)

        def gmf_gather_body(uq_vmem, iq_vmem, gu_vmem, gi_vmem):
            pltpu.sync_copy(gu_w_hbm.at[uq_vmem.at[0]], gu_vmem)
            pltpu.sync_copy(gi_w_hbm.at[iq_vmem.at[0]], gi_vmem)

        pltpu.emit_pipeline(
            gmf_gather_body,
            grid=(n // GATHER_WINDOW,),
            in_specs=[idx_spec, idx_spec],
            out_specs=[row_spec, row_spec],
            core_axis_name=("core", "subcore"),
            dimension_semantics=(pltpu.PARALLEL,),
        )(uq_hbm, iq_hbm, gu_hbm, gi_hbm)

    return gather_kernel(uq2, iq2, u2, i2, gmf_user_p, gmf_item_p,
                         mlp_user_w, mlp_item_w)


def _chunk_select(wide, rem_col):
    acc = None
    for c in range(PACK):
        mask = (rem_col == c).astype(jnp.float32)
        part = wide[:, c * EMB:(c + 1) * EMB] * mask
        acc = part if acc is None else acc + part
    return acc


def _mlp_body(ur_ref, ir_ref, gu_ref, gi_ref, mu_ref, mi_ref,
              w1u_ref, w1i_ref, b1_ref, w2_ref, b2_ref, w3_ref, b3_ref,
              wpg_ref, wph_ref, bp_ref, out_ref):
    h = jnp.dot(mu_ref[...], w1u_ref[...], preferred_element_type=jnp.float32)
    h = h + jnp.dot(mi_ref[...], w1i_ref[...], preferred_element_type=jnp.float32)
    h = jnp.maximum(h + b1_ref[...], 0.0)
    h = jnp.dot(h, w2_ref[...], preferred_element_type=jnp.float32) + b2_ref[...]
    h = jnp.maximum(h, 0.0)
    h = jnp.dot(h, w3_ref[...], preferred_element_type=jnp.float32) + b3_ref[...]
    h = jnp.maximum(h, 0.0)
    gu = _chunk_select(gu_ref[...], ur_ref[...])
    gi = _chunk_select(gi_ref[...], ir_ref[...])
    gmf = gu * gi
    out = jnp.dot(gmf, wpg_ref[...], preferred_element_type=jnp.float32)
    out = out + jnp.dot(h, wph_ref[...], preferred_element_type=jnp.float32)
    out_ref[...] = out + bp_ref[0, 0]


def _tc_mlp(ur, ir, gu, gi, mu, mi, W1, b1, W2, b2, W3, b3, Wp, bp):
    n = gu.shape[0]
    w1u = W1[:, :MLP_EMB].T
    w1i = W1[:, MLP_EMB:].T
    w2t = W2.T
    w3t = W3.T
    wpg = Wp[:, :EMB].T
    wph = Wp[:, EMB:].T
    b1r = b1.reshape(1, -1)
    b2r = b2.reshape(1, -1)
    b3r = b3.reshape(1, -1)
    bpr = bp.reshape(1, 1)

    full = lambda shape: pl.BlockSpec(shape, lambda i: (0, 0))
    col_spec = pl.BlockSpec((B_BLK, 1), lambda i: (i, 0))
    wide_spec = pl.BlockSpec((B_BLK, MLP_EMB), lambda i: (i, 0))
    out = pl.pallas_call(
        _mlp_body,
        grid=(n // B_BLK,),
        in_specs=[
            col_spec, col_spec,
            wide_spec, wide_spec, wide_spec, wide_spec,
            full(w1u.shape), full(w1i.shape), full(b1r.shape),
            full(w2t.shape), full(b2r.shape),
            full(w3t.shape), full(b3r.shape),
            full(wpg.shape), full(wph.shape), full(bpr.shape),
        ],
        out_specs=col_spec,
        out_shape=jax.ShapeDtypeStruct((n, 1), jnp.float32),
    )(ur, ir, gu, gi, mu, mi, w1u, w1i, b1r, w2t, b2r, w3t, b3r,
      wpg, wph, bpr)
    return out.reshape(-1)


def kernel(user, item, gmf_user_w, gmf_item_w, mlp_user_w, mlp_item_w,
           W1, b1, W2, b2, W3, b3, Wp, bp):
    n = user.shape[0]
    user = user.astype(jnp.int32)
    item = item.astype(jnp.int32)
    uq2 = (user // PACK).reshape(1, n)
    iq2 = (item // PACK).reshape(1, n)
    ur = (user % PACK).reshape(n, 1)
    ir = (item % PACK).reshape(n, 1)
    gmf_user_p = gmf_user_w.reshape(-1, MLP_EMB)
    gmf_item_p = gmf_item_w.reshape(-1, MLP_EMB)
    gu, gi, mu, mi = _sc_gather(uq2, iq2, user.reshape(1, n), item.reshape(1, n),
                                gmf_user_p, gmf_item_p, mlp_user_w, mlp_item_w)
    return _tc_mlp(ur, ir, gu, gi, mu, mi, W1, b1, W2, b2, W3, b3, Wp, bp)
